# baseline (speedup 1.0000x reference)
import jax
import jax.numpy as jnp
from jax import lax
from jax.experimental import pallas as pl
from jax.experimental.pallas import tpu as pltpu
from jax.experimental.pallas import tpu_sc as plsc

N = 10000
E = 320000
D = 128
L = 2

NC = 2
NS = 16
NW = NC * NS
EPT = E // NW
K = 128
NCHP = 80
EPT2 = NCHP * K
PAD = EPT2 - EPT
NP = 10240
RPT = NP // NS


def _sc_agg_body(h_hbm, meta_hbm, zero_hbm, out_hbm,
                 mb0, mb1, rows0, rows1, acc_sh, ms0, ms1, gs0, gs1):
    cid = lax.axis_index("c")
    sid = lax.axis_index("s")
    tid = cid * NS + sid
    mbufs = (mb0, mb1)
    msems = (ms0, ms1)
    rows = (rows0, rows1)
    gsems = (gs0, gs1)
    cbase = tid * NCHP

    pltpu.sync_copy(zero_hbm, acc_sh.at[pl.ds(sid * RPT, RPT)])
    plsc.subcore_barrier()

    def scale(p):
        rv = rows[p]
        wref = mbufs[p].at[2]

        @plsc.parallel_loop(0, K, unroll=8)
        def _(i):
            wb = plsc.bitcast(
                plsc.load_gather(wref, [jnp.full((16,), i, jnp.int32)]),
                jnp.float32)
            for j in range(D // 16):
                sl = pl.ds(j * 16, 16)
                rv[i, sl] = rv[i, sl] * wb

    def step(c, p, do_prev, do_next):
        q = (p + 1) % 2
        pltpu.make_async_copy(meta_hbm.at[cbase], mbufs[p], msems[p]).wait()
        gd = pltpu.async_copy(h_hbm.at[mbufs[p].at[0]], rows[p], gsems[p])
        if do_prev:
            scale(q)
            pltpu.sync_copy(rows[q], acc_sh.at[mbufs[q].at[1]], add=True)
        if do_next:
            pltpu.async_copy(meta_hbm.at[cbase + c + 1], mbufs[q], msems[q])
        gd.wait()

    pltpu.async_copy(meta_hbm.at[cbase], mb0, ms0)
    step(0, 0, do_prev=False, do_next=True)
    step(1, 1, do_prev=True, do_next=True)

    def body2(g, carry):
        step(2 * g, 0, do_prev=True, do_next=True)
        step(2 * g + 1, 1, do_prev=True, do_next=True)
        return carry

    lax.fori_loop(1, NCHP // 2 - 1, body2, 0)
    step(NCHP - 2, 0, do_prev=True, do_next=True)
    step(NCHP - 1, 1, do_prev=True, do_next=False)
    scale(1)
    pltpu.sync_copy(rows[1], acc_sh.at[mbufs[1].at[1]], add=True)

    plsc.subcore_barrier()
    row0 = cid * NP + sid * RPT
    pltpu.sync_copy(acc_sh.at[pl.ds(sid * RPT, RPT)],
                    out_hbm.at[pl.ds(row0, RPT)])


@jax.jit
def _sc_agg(h, meta, zero_rows):
    mesh = plsc.VectorSubcoreMesh(core_axis_name="c", subcore_axis_name="s")
    return pl.kernel(
        _sc_agg_body,
        out_type=jax.ShapeDtypeStruct((NC * NP, D), jnp.float32),
        mesh=mesh,
        scratch_types=[
            pltpu.VMEM((3, K), jnp.int32),
            pltpu.VMEM((3, K), jnp.int32),
            pltpu.VMEM((K, D), jnp.float32),
            pltpu.VMEM((K, D), jnp.float32),
            pltpu.VMEM_SHARED((NP, D), jnp.float32),
            pltpu.SemaphoreType.DMA,
            pltpu.SemaphoreType.DMA,
            pltpu.SemaphoreType.DMA,
            pltpu.SemaphoreType.DMA,
        ],
        compiler_params=pltpu.CompilerParams(use_tc_tiling_on_sc=False,
                                             needs_layout_passes=False),
    )(h, meta, zero_rows)


def _bn(x, g, b):
    m = jnp.mean(x, axis=0, keepdims=True)
    v = jnp.var(x, axis=0, keepdims=True)
    return (x - m) / jnp.sqrt(v + 1e-5) * g + b


def _dense_body(agg_ref, h_ref, W1_ref, b1_ref, g1_ref, be1_ref,
                W2_ref, b2_ref, g2_ref, be2_ref, g3_ref, be3_ref, out_ref):
    x = agg_ref[0] + agg_ref[1] + h_ref[...]
    x = jnp.dot(x, W1_ref[...].T, preferred_element_type=jnp.float32)
    x = x + b1_ref[...]
    x = jax.nn.relu(_bn(x, g1_ref[...], be1_ref[...]))
    x = jnp.dot(x, W2_ref[...].T, preferred_element_type=jnp.float32)
    x = x + b2_ref[...]
    x = jax.nn.relu(_bn(x, g2_ref[...], be2_ref[...]))
    out_ref[...] = jax.nn.relu(_bn(x, g3_ref[...], be3_ref[...]))


@jax.jit
def _dense(agg2, h, W1l, b1l, g1l, be1l, W2l, b2l, g2l, be2l, g3l, be3l):
    return pl.pallas_call(
        _dense_body,
        out_shape=jax.ShapeDtypeStruct((N, D), jnp.float32),
    )(agg2, h, W1l, b1l, g1l, be1l, W2l, b2l, g2l, be2l, g3l, be3l)


def kernel(h, edge_weight, W1, b1, g1, be1, W2, b2, g2, be2, g3, be3, edge_index):
    src = edge_index[0]
    dst = edge_index[1]
    zero_rows = jnp.zeros((RPT, D), jnp.float32)
    srcp = jnp.pad(src.reshape(NW, EPT), ((0, 0), (0, PAD))).reshape(NW, NCHP, K)
    dstp = jnp.pad(dst.reshape(NW, EPT), ((0, 0), (0, PAD))).reshape(NW, NCHP, K)
    for l in range(L):
        wi = lax.bitcast_convert_type(edge_weight[l], jnp.int32)
        wp = jnp.pad(wi.reshape(NW, EPT), ((0, 0), (0, PAD))).reshape(NW, NCHP, K)
        meta = jnp.stack([srcp, dstp, wp], axis=2).reshape(NW * NCHP, 3, K)
        agg = _sc_agg(h, meta, zero_rows)
        agg2 = agg.reshape(NC, NP, D)[:, :N]
        h = _dense(agg2, h,
                   W1[l], b1[l].reshape(1, D), g1[l].reshape(1, D),
                   be1[l].reshape(1, D), W2[l], b2[l].reshape(1, D),
                   g2[l].reshape(1, D), be2[l].reshape(1, D),
                   g3[l].reshape(1, D), be3[l].reshape(1, D))
    return h


# speedup vs baseline: 1.5208x; 1.5208x over previous
import jax
import jax.numpy as jnp
from jax import lax
from jax.experimental import pallas as pl
from jax.experimental.pallas import tpu as pltpu
from jax.experimental.pallas import tpu_sc as plsc

N = 10000
E = 320000
D = 128
L = 2

NC = 2
NS = 16
NW = NC * NS
EPT = E // NW
K = 96
NCHP = 105
EPT2 = NCHP * K
PAD = EPT2 - EPT
NP = 10240
RPT = NP // NS


def _sc_agg_body(h_hbm, meta_hbm, zero_hbm, out_hbm,
                 mb0, mb1, rows0, rows1, rows2, acc_sh, ms0, ms1, gsem):
    cid = lax.axis_index("c")
    sid = lax.axis_index("s")
    tid = cid * NS + sid
    mbufs = (mb0, mb1)
    msems = (ms0, ms1)
    rows = (rows0, rows1, rows2)
    cbase = tid * NCHP

    pltpu.sync_copy(zero_hbm, acc_sh.at[pl.ds(sid * RPT, RPT)])
    plsc.subcore_barrier()

    def scale(p3, mp):
        rv = rows[p3]
        wref = mbufs[mp].at[2]

        @plsc.parallel_loop(0, K, unroll=8)
        def _(i):
            wb = plsc.bitcast(
                plsc.load_gather(wref, [jnp.full((16,), i, jnp.int32)]),
                jnp.float32)
            for j in range(D // 16):
                sl = pl.ds(j * 16, 16)
                rv[i, sl] = rv[i, sl] * wb

    def step(c, p3, mp, has_next, has_next2):
        if has_next:
            q = (mp + 1) % 2
            pltpu.make_async_copy(meta_hbm.at[cbase], mbufs[q],
                                  msems[q]).wait()
            pltpu.async_copy(h_hbm.at[mbufs[q].at[0]], rows[(p3 + 1) % 3],
                             gsem)
        pltpu.make_async_copy(h_hbm.at[mbufs[mp].at[0]], rows[p3],
                              gsem).wait()
        scale(p3, mp)
        pltpu.sync_copy(rows[p3], acc_sh.at[mbufs[mp].at[1]], add=True)
        if has_next2:
            pltpu.async_copy(meta_hbm.at[cbase + c + 2], mbufs[mp],
                             msems[mp])

    pltpu.sync_copy(meta_hbm.at[cbase], mb0)
    pltpu.async_copy(h_hbm.at[mb0.at[0]], rows0, gsem)
    pltpu.async_copy(meta_hbm.at[cbase + 1], mb1, ms1)

    def body6(g, carry):
        for k in range(6):
            step(6 * g + k, k % 3, k % 2, has_next=True, has_next2=True)
        return carry

    nloop = ((NCHP - 2) // 6) * 6
    lax.fori_loop(0, nloop // 6, body6, 0)
    for c in range(nloop, NCHP):
        step(c, c % 3, c % 2, has_next=(c < NCHP - 1),
             has_next2=(c < NCHP - 2))

    plsc.subcore_barrier()
    row0 = cid * NP + sid * RPT
    pltpu.sync_copy(acc_sh.at[pl.ds(sid * RPT, RPT)],
                    out_hbm.at[pl.ds(row0, RPT)])


@jax.jit
def _sc_agg(h, meta, zero_rows):
    mesh = plsc.VectorSubcoreMesh(core_axis_name="c", subcore_axis_name="s")
    return pl.kernel(
        _sc_agg_body,
        out_type=jax.ShapeDtypeStruct((NC * NP, D), jnp.float32),
        mesh=mesh,
        scratch_types=[
            pltpu.VMEM((3, K), jnp.int32),
            pltpu.VMEM((3, K), jnp.int32),
            pltpu.VMEM((K, D), jnp.float32),
            pltpu.VMEM((K, D), jnp.float32),
            pltpu.VMEM((K, D), jnp.float32),
            pltpu.VMEM_SHARED((NP, D), jnp.float32),
            pltpu.SemaphoreType.DMA,
            pltpu.SemaphoreType.DMA,
            pltpu.SemaphoreType.DMA,
        ],
        compiler_params=pltpu.CompilerParams(use_tc_tiling_on_sc=False,
                                             needs_layout_passes=False),
    )(h, meta, zero_rows)


def _bn(x, g, b):
    m = jnp.mean(x, axis=0, keepdims=True)
    v = jnp.var(x, axis=0, keepdims=True)
    return (x - m) / jnp.sqrt(v + 1e-5) * g + b


def _dense_body(agg_ref, h_ref, W1_ref, b1_ref, g1_ref, be1_ref,
                W2_ref, b2_ref, g2_ref, be2_ref, g3_ref, be3_ref, out_ref):
    x = agg_ref[0] + agg_ref[1] + h_ref[...]
    x = jnp.dot(x, W1_ref[...].T, preferred_element_type=jnp.float32)
    x = x + b1_ref[...]
    x = jax.nn.relu(_bn(x, g1_ref[...], be1_ref[...]))
    x = jnp.dot(x, W2_ref[...].T, preferred_element_type=jnp.float32)
    x = x + b2_ref[...]
    x = jax.nn.relu(_bn(x, g2_ref[...], be2_ref[...]))
    out_ref[...] = jax.nn.relu(_bn(x, g3_ref[...], be3_ref[...]))


@jax.jit
def _dense(agg2, h, W1l, b1l, g1l, be1l, W2l, b2l, g2l, be2l, g3l, be3l):
    return pl.pallas_call(
        _dense_body,
        out_shape=jax.ShapeDtypeStruct((N, D), jnp.float32),
    )(agg2, h, W1l, b1l, g1l, be1l, W2l, b2l, g2l, be2l, g3l, be3l)


def kernel(h, edge_weight, W1, b1, g1, be1, W2, b2, g2, be2, g3, be3, edge_index):
    src = edge_index[0]
    dst = edge_index[1]
    zero_rows = jnp.zeros((RPT, D), jnp.float32)
    srcp = jnp.pad(src.reshape(NW, EPT), ((0, 0), (0, PAD))).reshape(NW, NCHP, K)
    dstp = jnp.pad(dst.reshape(NW, EPT), ((0, 0), (0, PAD))).reshape(NW, NCHP, K)
    for l in range(L):
        wi = lax.bitcast_convert_type(edge_weight[l], jnp.int32)
        wp = jnp.pad(wi.reshape(NW, EPT), ((0, 0), (0, PAD))).reshape(NW, NCHP, K)
        meta = jnp.stack([srcp, dstp, wp], axis=2).reshape(NW * NCHP, 3, K)
        agg = _sc_agg(h, meta, zero_rows)
        agg2 = agg.reshape(NC, NP, D)[:, :N]
        h = _dense(agg2, h,
                   W1[l], b1[l].reshape(1, D), g1[l].reshape(1, D),
                   be1[l].reshape(1, D), W2[l], b2[l].reshape(1, D),
                   g2[l].reshape(1, D), be2[l].reshape(1, D),
                   g3[l].reshape(1, D), be3[l].reshape(1, D))
    return h


# speedup vs baseline: 1.8181x; 1.1954x over previous
import jax
import jax.numpy as jnp
from jax import lax
from jax.experimental import pallas as pl
from jax.experimental.pallas import tpu as pltpu
from jax.experimental.pallas import tpu_sc as plsc

N = 10000
E = 320000
D = 128
L = 2

NC = 2
NS = 16
NW = NC * NS
EPT = E // NW
K = 88
NCHP = 114
GD = 3
RB = GD + 1
EPT2 = NCHP * K
PAD = EPT2 - EPT
NP = 10240
RPT = NP // NS


def _sc_agg_body(h_hbm, meta_hbm, zero_hbm, out_hbm,
                 mb0, mb1, mb2, mb3, rows0, rows1, rows2, rows3, acc_sh,
                 ms0, ms1, ms2, ms3, gsem):
    cid = lax.axis_index("c")
    sid = lax.axis_index("s")
    tid = cid * NS + sid
    mbufs = (mb0, mb1, mb2, mb3)
    msems = (ms0, ms1, ms2, ms3)
    rows = (rows0, rows1, rows2, rows3)
    cbase = tid * NCHP

    pltpu.sync_copy(zero_hbm, acc_sh.at[pl.ds(sid * RPT, RPT)])
    plsc.subcore_barrier()

    def scale(r):
        rv = rows[r]
        wref = mbufs[r].at[2]

        @plsc.parallel_loop(0, K, unroll=8)
        def _(i):
            wb = plsc.bitcast(
                plsc.load_gather(wref, [jnp.full((16,), i, jnp.int32)]),
                jnp.float32)
            for j in range(D // 16):
                sl = pl.ds(j * 16, 16)
                rv[i, sl] = rv[i, sl] * wb

    def step(c, r, has_next, has_next2):
        if has_next:
            rn = (r + GD) % RB
            pltpu.make_async_copy(meta_hbm.at[cbase], mbufs[rn],
                                  msems[rn]).wait()
            pltpu.async_copy(h_hbm.at[mbufs[rn].at[0]], rows[rn], gsem)
        pltpu.make_async_copy(h_hbm.at[mbufs[r].at[0]], rows[r],
                              gsem).wait()
        scale(r)
        pltpu.sync_copy(rows[r], acc_sh.at[mbufs[r].at[1]], add=True)
        if has_next2:
            pltpu.async_copy(meta_hbm.at[cbase + c + GD + 1], mbufs[r],
                             msems[r])

    for j in range(GD):
        pltpu.sync_copy(meta_hbm.at[cbase + j], mbufs[j])
        pltpu.async_copy(h_hbm.at[mbufs[j].at[0]], rows[j], gsem)
    pltpu.async_copy(meta_hbm.at[cbase + GD], mbufs[GD], msems[GD])

    def bodyn(g, carry):
        for k in range(RB):
            step(RB * g + k, k, has_next=True, has_next2=True)
        return carry

    nloop = RB * ((NCHP - GD - 1) // RB)
    lax.fori_loop(0, nloop // RB, bodyn, 0)
    for c in range(nloop, NCHP):
        step(c, c % RB, has_next=(c + GD < NCHP),
             has_next2=(c + GD + 1 < NCHP))

    plsc.subcore_barrier()
    row0 = cid * NP + sid * RPT
    pltpu.sync_copy(acc_sh.at[pl.ds(sid * RPT, RPT)],
                    out_hbm.at[pl.ds(row0, RPT)])


@jax.jit
def _sc_agg(h, meta, zero_rows):
    mesh = plsc.VectorSubcoreMesh(core_axis_name="c", subcore_axis_name="s")
    return pl.kernel(
        _sc_agg_body,
        out_type=jax.ShapeDtypeStruct((NC * NP, D), jnp.float32),
        mesh=mesh,
        scratch_types=[
            pltpu.VMEM((3, K), jnp.int32),
            pltpu.VMEM((3, K), jnp.int32),
            pltpu.VMEM((3, K), jnp.int32),
            pltpu.VMEM((3, K), jnp.int32),
            pltpu.VMEM((K, D), jnp.float32),
            pltpu.VMEM((K, D), jnp.float32),
            pltpu.VMEM((K, D), jnp.float32),
            pltpu.VMEM((K, D), jnp.float32),
            pltpu.VMEM_SHARED((NP, D), jnp.float32),
            pltpu.SemaphoreType.DMA,
            pltpu.SemaphoreType.DMA,
            pltpu.SemaphoreType.DMA,
            pltpu.SemaphoreType.DMA,
            pltpu.SemaphoreType.DMA,
        ],
        compiler_params=pltpu.CompilerParams(use_tc_tiling_on_sc=False,
                                             needs_layout_passes=False),
    )(h, meta, zero_rows)


def _bn(x, g, b):
    m = jnp.mean(x, axis=0, keepdims=True)
    v = jnp.var(x, axis=0, keepdims=True)
    return (x - m) / jnp.sqrt(v + 1e-5) * g + b


def _dense_body(agg_ref, h_ref, W1_ref, b1_ref, g1_ref, be1_ref,
                W2_ref, b2_ref, g2_ref, be2_ref, g3_ref, be3_ref, out_ref):
    x = agg_ref[0] + agg_ref[1] + h_ref[...]
    x = jnp.dot(x, W1_ref[...].T, preferred_element_type=jnp.float32)
    x = x + b1_ref[...]
    x = jax.nn.relu(_bn(x, g1_ref[...], be1_ref[...]))
    x = jnp.dot(x, W2_ref[...].T, preferred_element_type=jnp.float32)
    x = x + b2_ref[...]
    x = jax.nn.relu(_bn(x, g2_ref[...], be2_ref[...]))
    out_ref[...] = jax.nn.relu(_bn(x, g3_ref[...], be3_ref[...]))


@jax.jit
def _dense(agg2, h, W1l, b1l, g1l, be1l, W2l, b2l, g2l, be2l, g3l, be3l):
    return pl.pallas_call(
        _dense_body,
        out_shape=jax.ShapeDtypeStruct((N, D), jnp.float32),
    )(agg2, h, W1l, b1l, g1l, be1l, W2l, b2l, g2l, be2l, g3l, be3l)


def kernel(h, edge_weight, W1, b1, g1, be1, W2, b2, g2, be2, g3, be3, edge_index):
    src = edge_index[0]
    dst = edge_index[1]
    zero_rows = jnp.zeros((RPT, D), jnp.float32)
    srcp = jnp.pad(src.reshape(NW, EPT), ((0, 0), (0, PAD))).reshape(NW, NCHP, K)
    dstp = jnp.pad(dst.reshape(NW, EPT), ((0, 0), (0, PAD))).reshape(NW, NCHP, K)
    for l in range(L):
        wi = lax.bitcast_convert_type(edge_weight[l], jnp.int32)
        wp = jnp.pad(wi.reshape(NW, EPT), ((0, 0), (0, PAD))).reshape(NW, NCHP, K)
        meta = jnp.stack([srcp, dstp, wp], axis=2).reshape(NW * NCHP, 3, K)
        agg = _sc_agg(h, meta, zero_rows)
        agg2 = agg.reshape(NC, NP, D)[:, :N]
        h = _dense(agg2, h,
                   W1[l], b1[l].reshape(1, D), g1[l].reshape(1, D),
                   be1[l].reshape(1, D), W2[l], b2[l].reshape(1, D),
                   g2[l].reshape(1, D), be2[l].reshape(1, D),
                   g3[l].reshape(1, D), be3[l].reshape(1, D))
    return h
